# baseline (speedup 1.0000x reference)
import jax
import jax.numpy as jnp
from jax.experimental import pallas as pl
from jax.experimental.pallas import tpu as pltpu


def tc_gather(idx, emb):
    b = idx.shape[0]

    def body(idx_ref, in_ref, out_ref):
        out_ref[...] = in_ref[...]

    grid_spec = pltpu.PrefetchScalarGridSpec(
        num_scalar_prefetch=1,
        grid=(b,),
        in_specs=[
            pl.BlockSpec((1, 77, 4096), lambda i, idx_ref: (idx_ref[i], 0, 0)),
        ],
        out_specs=pl.BlockSpec((1, 77, 4096), lambda i, idx_ref: (i, 0, 0)),
    )
    return pl.pallas_call(
        body,
        grid_spec=grid_spec,
        out_shape=jax.ShapeDtypeStruct((b, 77, 4096), jnp.float32),
    )(idx, emb)


def kernel(prompt_idx, embeddings):
    return tc_gather(prompt_idx.astype(jnp.int32), embeddings)
